# baseline (speedup 1.0000x reference)
import functools

import jax
import jax.numpy as jnp
from jax.experimental import pallas as pl
from jax.experimental.pallas import tpu as pltpu

B, S, D, V = 4, 2048, 2048, 1000
M = B * S
TILE_M = 256
NUM_TILES = M // TILE_M


def _head_kernel(x_ref, w_ref, b_ref, tgt_ref, logits_ref, probs_ref, loss_ref,
                 acc_ref):
    i = pl.program_id(0)

    x = x_ref[...].astype(jnp.bfloat16)
    w = w_ref[...].astype(jnp.bfloat16)
    logits = jax.lax.dot_general(
        x, w, (((1,), (1,)), ((), ())),
        preferred_element_type=jnp.float32)
    logits = logits + b_ref[...]
    logits_ref[...] = logits

    m = jnp.max(logits, axis=-1, keepdims=True)
    ex = jnp.exp(logits - m)
    s = jnp.sum(ex, axis=-1, keepdims=True)
    probs_ref[...] = ex * (1.0 / s)

    t = tgt_ref[0, pl.ds(i * TILE_M, TILE_M)]
    t2 = t[:, None]
    onehot = (jax.lax.broadcasted_iota(jnp.int32, (TILE_M, V), 1) == t2)
    tgt_logit = jnp.sum(jnp.where(onehot, logits, 0.0), axis=-1, keepdims=True)
    lse = m + jnp.log(s)
    valid = t2 >= 0
    nll = jnp.where(valid, lse - tgt_logit, 0.0)

    tile_sum = jnp.sum(nll)
    tile_cnt = jnp.sum(valid.astype(jnp.float32))

    @pl.when(i == 0)
    def _init():
        acc_ref[0] = 0.0
        acc_ref[1] = 0.0

    acc_ref[0] += tile_sum
    acc_ref[1] += tile_cnt

    @pl.when(i == NUM_TILES - 1)
    def _fin():
        val = acc_ref[0] / jnp.maximum(acc_ref[1], 1.0)
        loss_ref[...] = jnp.broadcast_to(val, (1, 1))


@jax.jit
def _head(x, w, b, tgt):
    logits, probs, loss = pl.pallas_call(
        _head_kernel,
        grid=(NUM_TILES,),
        in_specs=[
            pl.BlockSpec((TILE_M, D), lambda i: (i, 0)),
            pl.BlockSpec((V, D), lambda i: (0, 0)),
            pl.BlockSpec((1, V), lambda i: (0, 0)),
            pl.BlockSpec((1, M), lambda i: (0, 0)),
        ],
        out_specs=[
            pl.BlockSpec((TILE_M, V), lambda i: (i, 0)),
            pl.BlockSpec((TILE_M, V), lambda i: (i, 0)),
            pl.BlockSpec((1, 1), lambda i: (0, 0)),
        ],
        out_shape=[
            jax.ShapeDtypeStruct((M, V), jnp.float32),
            jax.ShapeDtypeStruct((M, V), jnp.float32),
            jax.ShapeDtypeStruct((1, 1), jnp.float32),
        ],
        scratch_shapes=[pltpu.SMEM((2,), jnp.float32)],
    )(x, w, b, tgt)
    return logits, probs, loss


def kernel(encoder_out, target, target_mask, W, b):
    x = encoder_out.reshape(M, D)
    tgt = jnp.where(target_mask, target, -1).astype(jnp.int32).reshape(1, M)
    logits, probs, loss = _head(x, W, b.reshape(1, V), tgt)
    return (logits.reshape(B, S, V), probs.reshape(B, S, V), loss[0, 0])


# speedup vs baseline: 1.1161x; 1.1161x over previous
import functools

import jax
import jax.numpy as jnp
from jax.experimental import pallas as pl
from jax.experimental.pallas import tpu as pltpu

B, S, D, V = 4, 2048, 2048, 1000
M = B * S
TILE_M = 1024
NUM_TILES = M // TILE_M


def _head_kernel(x_ref, w_ref, b_ref, tgt_ref, logits_ref, probs_ref, loss_ref,
                 acc_ref):
    i = pl.program_id(0)

    x = x_ref[...].astype(jnp.bfloat16)
    w = w_ref[...].astype(jnp.bfloat16)
    logits = jax.lax.dot_general(
        x, w, (((1,), (1,)), ((), ())),
        preferred_element_type=jnp.float32)
    logits = logits + b_ref[...]
    logits_ref[...] = logits

    m = jnp.max(logits, axis=-1, keepdims=True)
    ex = jnp.exp(logits - m)
    s = jnp.sum(ex, axis=-1, keepdims=True)
    probs_ref[...] = ex * (1.0 / s)

    t = tgt_ref[0, pl.ds(i * TILE_M, TILE_M)]
    t2 = t[:, None]
    onehot = (jax.lax.broadcasted_iota(jnp.int32, (TILE_M, V), 1) == t2)
    tgt_logit = jnp.sum(jnp.where(onehot, logits, 0.0), axis=-1, keepdims=True)
    lse = m + jnp.log(s)
    valid = t2 >= 0
    nll = jnp.where(valid, lse - tgt_logit, 0.0)

    tile_sum = jnp.sum(nll)
    tile_cnt = jnp.sum(valid.astype(jnp.float32))

    @pl.when(i == 0)
    def _init():
        acc_ref[0] = 0.0
        acc_ref[1] = 0.0

    acc_ref[0] += tile_sum
    acc_ref[1] += tile_cnt

    @pl.when(i == NUM_TILES - 1)
    def _fin():
        val = acc_ref[0] / jnp.maximum(acc_ref[1], 1.0)
        loss_ref[...] = jnp.broadcast_to(val, (1, 1))


@jax.jit
def _head(x, w, b, tgt):
    logits, probs, loss = pl.pallas_call(
        _head_kernel,
        grid=(NUM_TILES,),
        in_specs=[
            pl.BlockSpec((TILE_M, D), lambda i: (i, 0)),
            pl.BlockSpec((V, D), lambda i: (0, 0)),
            pl.BlockSpec((1, V), lambda i: (0, 0)),
            pl.BlockSpec((1, M), lambda i: (0, 0)),
        ],
        out_specs=[
            pl.BlockSpec((TILE_M, V), lambda i: (i, 0)),
            pl.BlockSpec((TILE_M, V), lambda i: (i, 0)),
            pl.BlockSpec((1, 1), lambda i: (0, 0)),
        ],
        out_shape=[
            jax.ShapeDtypeStruct((M, V), jnp.float32),
            jax.ShapeDtypeStruct((M, V), jnp.float32),
            jax.ShapeDtypeStruct((1, 1), jnp.float32),
        ],
        scratch_shapes=[pltpu.SMEM((2,), jnp.float32)],
    )(x, w, b, tgt)
    return logits, probs, loss


def kernel(encoder_out, target, target_mask, W, b):
    x = encoder_out.reshape(M, D)
    tgt = jnp.where(target_mask, target, -1).astype(jnp.int32).reshape(1, M)
    logits, probs, loss = _head(x, W, b.reshape(1, V), tgt)
    return (logits.reshape(B, S, V), probs.reshape(B, S, V), loss[0, 0])


# speedup vs baseline: 1.1354x; 1.0173x over previous
import jax
import jax.numpy as jnp
from jax.experimental import pallas as pl
from jax.experimental.pallas import tpu as pltpu

B, S, D, V = 4, 2048, 2048, 1000
TILE_S = 1024
NS = S // TILE_S


def _head_kernel(x_ref, w_ref, b_ref, tgt_ref, logits_ref, probs_ref, loss_ref,
                 acc_ref):
    bi = pl.program_id(0)
    sj = pl.program_id(1)

    x = x_ref[0].astype(jnp.bfloat16)
    w = w_ref[...].astype(jnp.bfloat16)
    logits = jax.lax.dot_general(
        x, w, (((1,), (1,)), ((), ())),
        preferred_element_type=jnp.float32)
    logits = logits + b_ref[...]
    logits_ref[0] = logits

    m = jnp.max(logits, axis=-1, keepdims=True)
    ex = jnp.exp(logits - m)
    s = jnp.sum(ex, axis=-1, keepdims=True)
    probs_ref[0] = ex * (1.0 / s)

    t = tgt_ref[bi, pl.ds(sj * TILE_S, TILE_S)]
    t2 = t[:, None]
    onehot = (jax.lax.broadcasted_iota(jnp.int32, (TILE_S, V), 1) == t2)
    tgt_logit = jnp.sum(jnp.where(onehot, logits, 0.0), axis=-1, keepdims=True)
    lse = m + jnp.log(s)
    valid = t2 >= 0
    nll = jnp.where(valid, lse - tgt_logit, 0.0)

    tile_sum = jnp.sum(nll)
    tile_cnt = jnp.sum(valid.astype(jnp.float32))

    @pl.when((bi == 0) & (sj == 0))
    def _init():
        acc_ref[0] = 0.0
        acc_ref[1] = 0.0

    acc_ref[0] += tile_sum
    acc_ref[1] += tile_cnt

    @pl.when((bi == B - 1) & (sj == NS - 1))
    def _fin():
        val = acc_ref[0] / jnp.maximum(acc_ref[1], 1.0)
        loss_ref[...] = jnp.broadcast_to(val, (1, 1))


@jax.jit
def _head(x, w, b, tgt):
    logits, probs, loss = pl.pallas_call(
        _head_kernel,
        grid=(B, NS),
        in_specs=[
            pl.BlockSpec((1, TILE_S, D), lambda i, j: (i, j, 0)),
            pl.BlockSpec((V, D), lambda i, j: (0, 0)),
            pl.BlockSpec((1, V), lambda i, j: (0, 0)),
            pl.BlockSpec((B, S), lambda i, j: (0, 0)),
        ],
        out_specs=[
            pl.BlockSpec((1, TILE_S, V), lambda i, j: (i, j, 0)),
            pl.BlockSpec((1, TILE_S, V), lambda i, j: (i, j, 0)),
            pl.BlockSpec((1, 1), lambda i, j: (0, 0)),
        ],
        out_shape=[
            jax.ShapeDtypeStruct((B, S, V), jnp.float32),
            jax.ShapeDtypeStruct((B, S, V), jnp.float32),
            jax.ShapeDtypeStruct((1, 1), jnp.float32),
        ],
        scratch_shapes=[pltpu.SMEM((2,), jnp.float32)],
    )(x, w, b, tgt)
    return logits, probs, loss[0, 0]


def kernel(encoder_out, target, target_mask, W, b):
    tgt = jnp.where(target_mask, target, -1).astype(jnp.int32)
    return _head(encoder_out, W, b.reshape(1, V), tgt)


# speedup vs baseline: 2.2728x; 2.0018x over previous
import jax
import jax.numpy as jnp
from jax.experimental import pallas as pl
from jax.experimental.pallas import tpu as pltpu

B, S, D, V = 4, 2048, 2048, 1000
TILE_S = 1024
NS = S // TILE_S


def _head_kernel(x_ref, w_ref, b_ref, tgt_ref, logits_ref, probs_ref, loss_ref,
                 acc_ref):
    bi = pl.program_id(0)
    sj = pl.program_id(1)

    x = x_ref[0].astype(jnp.bfloat16)
    w = w_ref[...].astype(jnp.bfloat16)
    logits_t = jax.lax.dot_general(
        w, x, (((1,), (1,)), ((), ())),
        preferred_element_type=jnp.float32)
    logits_t = logits_t + b_ref[...]
    logits_ref[0] = logits_t

    m = jnp.max(logits_t, axis=0, keepdims=True)
    ex = jnp.exp(logits_t - m)
    s = jnp.sum(ex, axis=0, keepdims=True)
    probs_ref[0] = ex * (1.0 / s)

    t = tgt_ref[bi, pl.ds(sj * TILE_S, TILE_S)][None, :]
    onehot = (jax.lax.broadcasted_iota(jnp.int32, (V, TILE_S), 0) == t)
    tgt_logit = jnp.sum(jnp.where(onehot, logits_t, 0.0), axis=0, keepdims=True)
    lse = m + jnp.log(s)
    valid = t >= 0
    nll = jnp.where(valid, lse - tgt_logit, 0.0)

    tile_sum = jnp.sum(nll)
    tile_cnt = jnp.sum(valid.astype(jnp.float32))

    @pl.when((bi == 0) & (sj == 0))
    def _init():
        acc_ref[0] = 0.0
        acc_ref[1] = 0.0

    acc_ref[0] += tile_sum
    acc_ref[1] += tile_cnt

    @pl.when((bi == B - 1) & (sj == NS - 1))
    def _fin():
        val = acc_ref[0] / jnp.maximum(acc_ref[1], 1.0)
        loss_ref[...] = jnp.broadcast_to(val, (1, 1))


@jax.jit
def _head(x, w, b, tgt):
    logits_t, probs_t, loss = pl.pallas_call(
        _head_kernel,
        grid=(B, NS),
        in_specs=[
            pl.BlockSpec((1, TILE_S, D), lambda i, j: (i, j, 0)),
            pl.BlockSpec((V, D), lambda i, j: (0, 0)),
            pl.BlockSpec((V, 1), lambda i, j: (0, 0)),
            pl.BlockSpec((B, S), lambda i, j: (0, 0)),
        ],
        out_specs=[
            pl.BlockSpec((1, V, TILE_S), lambda i, j: (i, 0, j)),
            pl.BlockSpec((1, V, TILE_S), lambda i, j: (i, 0, j)),
            pl.BlockSpec((1, 1), lambda i, j: (0, 0)),
        ],
        out_shape=[
            jax.ShapeDtypeStruct((B, V, S), jnp.float32),
            jax.ShapeDtypeStruct((B, V, S), jnp.float32),
            jax.ShapeDtypeStruct((1, 1), jnp.float32),
        ],
        scratch_shapes=[pltpu.SMEM((2,), jnp.float32)],
    )(x, w, b, tgt)
    return (jnp.swapaxes(logits_t, 1, 2), jnp.swapaxes(probs_t, 1, 2),
            loss[0, 0])


def kernel(encoder_out, target, target_mask, W, b):
    tgt = jnp.where(target_mask, target, -1).astype(jnp.int32)
    return _head(encoder_out, W, b.reshape(V, 1), tgt)
